# baseline (speedup 1.0000x reference)
import functools

import jax
import jax.numpy as jnp
from jax import lax
from jax.experimental import pallas as pl
from jax.experimental.pallas import tpu as pltpu
from jax.experimental.pallas import tpu_sc as plsc

F32 = jnp.float32

NC = 2
NS = 16
NW = NC * NS

CHUNK = 128
HCOLS = 80


def _linear(x, w, b, relu):
    n, d = x.shape
    dout = w.shape[1]
    blk = 1000

    def body(x_ref, w_ref, b_ref, o_ref):
        y = jnp.dot(x_ref[...], w_ref[...], preferred_element_type=F32)
        y = y + b_ref[...]
        if relu:
            y = jnp.maximum(y, 0.0)
        o_ref[...] = y

    return pl.pallas_call(
        body,
        grid=(n // blk,),
        in_specs=[
            pl.BlockSpec((blk, d), lambda i: (i, 0)),
            pl.BlockSpec((d, dout), lambda i: (0, 0)),
            pl.BlockSpec((1, dout), lambda i: (0, 0)),
        ],
        out_specs=pl.BlockSpec((blk, dout), lambda i: (i, 0)),
        out_shape=jax.ShapeDtypeStruct((n, dout), F32),
    )(x, w, b.reshape(1, dout))


def _linear_chain(x, w1, b1, w2, b2):
    n, d = x.shape
    dout = w2.shape[1]
    blk = 1000

    def body(x_ref, w1_ref, b1_ref, w2_ref, b2_ref, o_ref):
        y = jnp.dot(x_ref[...], w1_ref[...], preferred_element_type=F32)
        y = jnp.maximum(y + b1_ref[...], 0.0)
        o_ref[...] = jnp.dot(y, w2_ref[...],
                             preferred_element_type=F32) + b2_ref[...]

    h = w1.shape[1]
    return pl.pallas_call(
        body,
        grid=(n // blk,),
        in_specs=[
            pl.BlockSpec((blk, d), lambda i: (i, 0)),
            pl.BlockSpec((d, h), lambda i: (0, 0)),
            pl.BlockSpec((1, h), lambda i: (0, 0)),
            pl.BlockSpec((h, dout), lambda i: (0, 0)),
            pl.BlockSpec((1, dout), lambda i: (0, 0)),
        ],
        out_specs=pl.BlockSpec((blk, dout), lambda i: (i, 0)),
        out_shape=jax.ShapeDtypeStruct((n, dout), F32),
    )(x, w1, b1.reshape(1, h), w2, b2.reshape(1, dout))


def _linear_dual(x, wa, ba, wb, bb):
    n, d = x.shape
    dout = wa.shape[1]
    blk = 1000

    def body(x_ref, wa_ref, ba_ref, wb_ref, bb_ref, oa_ref, ob_ref):
        xv = x_ref[...]
        oa_ref[...] = jnp.dot(xv, wa_ref[...],
                              preferred_element_type=F32) + ba_ref[...]
        ob_ref[...] = jnp.dot(xv, wb_ref[...],
                              preferred_element_type=F32) + bb_ref[...]

    return pl.pallas_call(
        body,
        grid=(n // blk,),
        in_specs=[
            pl.BlockSpec((blk, d), lambda i: (i, 0)),
            pl.BlockSpec((d, dout), lambda i: (0, 0)),
            pl.BlockSpec((1, dout), lambda i: (0, 0)),
            pl.BlockSpec((d, dout), lambda i: (0, 0)),
            pl.BlockSpec((1, dout), lambda i: (0, 0)),
        ],
        out_specs=[
            pl.BlockSpec((blk, dout), lambda i: (i, 0)),
            pl.BlockSpec((blk, dout), lambda i: (i, 0)),
        ],
        out_shape=[
            jax.ShapeDtypeStruct((n, dout), F32),
            jax.ShapeDtypeStruct((n, dout), F32),
        ],
    )(x, wa, ba.reshape(1, dout), wb, bb.reshape(1, dout))


def _edge_scores(m_g3, att):
    eb = m_g3.shape[0]
    rb = 10
    g = eb // rb

    def body(m_ref, att_ref, e_ref, bm_ref):
        m = m_ref[...]
        m = jnp.where(m > 0, m, 0.2 * m)
        a = att_ref[0, :]
        e = lax.dot_general(m, a, (((2,), (0,)), ((), ())),
                            preferred_element_type=F32)
        e_ref[...] = e[None]
        bm_ref[...] = jnp.full((1, 1, 128), jnp.max(e), F32)

    return pl.pallas_call(
        body,
        grid=(g,),
        in_specs=[
            pl.BlockSpec((rb, 128, 128), lambda i: (i, 0, 0)),
            pl.BlockSpec((1, 128), lambda i: (0, 0)),
        ],
        out_specs=[
            pl.BlockSpec((1, rb, 128), lambda i: (i, 0, 0)),
            pl.BlockSpec((1, 1, 128), lambda i: (i, 0, 0)),
        ],
        out_shape=[
            jax.ShapeDtypeStruct((g, rb, 128), F32),
            jax.ShapeDtypeStruct((g, 1, 128), F32),
        ],
    )(m_g3, att.reshape(1, 128))


def _edge_weights(e2, bmax, hs_g3, ebp):
    eb = hs_g3.shape[0]
    rb = 10
    g = eb // rb

    def body(e_ref, bm_ref, hs_ref, wa_ref):
        mglob = jnp.max(bm_ref[...])
        ex = jnp.exp(e_ref[0] - mglob)
        hs = hs_ref[...]
        exn = ex[:, :, None]
        w = hs * exn
        z16 = jnp.zeros((rb, 128, 16), F32)
        i16 = lax.broadcasted_iota(jnp.int32, (rb, 128, 16), 2)
        x16 = jnp.where(i16 == 0, exn, 0.0)
        wa_ref[0] = jnp.concatenate([w[:, :, :64], z16], axis=2)
        wa_ref[1] = jnp.concatenate([w[:, :, 64:], x16], axis=2)

    return pl.pallas_call(
        body,
        grid=(g,),
        in_specs=[
            pl.BlockSpec((1, rb, 128), lambda i: (i, 0, 0)),
            pl.BlockSpec((g, 1, 128), lambda i: (0, 0, 0)),
            pl.BlockSpec((rb, 128, 128), lambda i: (i, 0, 0)),
        ],
        out_specs=pl.BlockSpec((2, rb, 128, HCOLS), lambda i: (0, i, 0, 0)),
        out_shape=jax.ShapeDtypeStruct((2, ebp, 128, HCOLS), F32),
    )(e2, bmax, hs_g3)


def _finalize_linear(p, bias, w, b, n):
    blk = 1000
    dout = w.shape[1]

    def body(p_ref, bias_ref, w_ref, b_ref, o_ref):
        lo = p_ref[0]
        hi = p_ref[1]
        u = jnp.concatenate([lo[:, :64], hi[:, :64]], axis=1)
        den = hi[:, 64:65]
        y = u / (den + 1e-16) + bias_ref[...]
        x = jnp.maximum(y, 0.0)
        o_ref[...] = jnp.dot(x, w_ref[...],
                             preferred_element_type=F32) + b_ref[...]

    return pl.pallas_call(
        body,
        grid=(n // blk,),
        in_specs=[
            pl.BlockSpec((2, blk, HCOLS), lambda i: (0, i, 0)),
            pl.BlockSpec((1, 128), lambda i: (0, 0)),
            pl.BlockSpec((128, dout), lambda i: (0, 0)),
            pl.BlockSpec((1, dout), lambda i: (0, 0)),
        ],
        out_specs=pl.BlockSpec((blk, dout), lambda i: (i, 0)),
        out_shape=jax.ShapeDtypeStruct((n, dout), F32),
    )(p, bias.reshape(1, 128), w, b.reshape(1, dout))


def _sc_gather2(hs, hd, src, dst):
    n, h = hs.shape
    e = src.shape[0]
    dt = hs.dtype
    nchunks = e // CHUNK
    nju = (nchunks + NW - 1) // NW
    nju = nju + (nju & 1)
    npairs = nju // 2
    mesh = plsc.VectorSubcoreMesh(core_axis_name="c", subcore_axis_name="s")

    @functools.partial(
        pl.kernel,
        out_type=(jax.ShapeDtypeStruct((e, h), dt),
                  jax.ShapeDtypeStruct((e, h), dt)),
        mesh=mesh,
        scratch_types=[
            pltpu.VMEM((CHUNK,), jnp.int32),
            pltpu.VMEM((CHUNK,), jnp.int32),
            pltpu.VMEM((CHUNK,), jnp.int32),
            pltpu.VMEM((CHUNK,), jnp.int32),
            pltpu.VMEM((CHUNK, h), dt),
            pltpu.VMEM((CHUNK, h), dt),
            pltpu.VMEM((CHUNK, h), dt),
            pltpu.VMEM((CHUNK, h), dt),
            pltpu.SemaphoreType.DMA,
            pltpu.SemaphoreType.DMA,
            pltpu.SemaphoreType.DMA,
            pltpu.SemaphoreType.DMA,
        ],
        compiler_params=pltpu.CompilerParams(use_tc_tiling_on_sc=False),
    )
    def k(hs_hbm, hd_hbm, src_hbm, dst_hbm, ohs_hbm, ohd_hbm,
          si0, si1, di0, di1, hsb0, hsb1, hdb0, hdb1, sg0, sg1, sw0, sw1):
        wid = lax.axis_index("s") * NC + lax.axis_index("c")
        si = (si0, si1)
        di = (di0, di1)
        hsb = (hsb0, hsb1)
        hdb = (hdb0, hdb1)
        sg = (sg0, sg1)
        sw = (sw0, sw1)
        nj = (nchunks - wid + NW - 1) // NW

        def off(j):
            jc = jnp.minimum(j, nj - 1)
            return (wid + jc * NW) * CHUNK

        def load_idx(j, p):
            pltpu.sync_copy(src_hbm.at[pl.ds(off(j), CHUNK)], si[p])
            pltpu.sync_copy(dst_hbm.at[pl.ds(off(j), CHUNK)], di[p])

        def start_g(p):
            pltpu.async_copy(hs_hbm.at[si[p]], hsb[p], sg[p])
            pltpu.async_copy(hd_hbm.at[di[p]], hdb[p], sg[p])

        def drain_g(p):
            pltpu.make_async_copy(hs_hbm.at[si[p]], hsb[p], sg[p]).wait()
            pltpu.make_async_copy(hd_hbm.at[di[p]], hdb[p], sg[p]).wait()

        def start_w(j, p):
            pltpu.async_copy(hsb[p], ohs_hbm.at[pl.ds(off(j), CHUNK)], sw[p])
            pltpu.async_copy(hdb[p], ohd_hbm.at[pl.ds(off(j), CHUNK)], sw[p])

        def drain_w(j, p):
            pltpu.make_async_copy(
                hsb[p], ohs_hbm.at[pl.ds(off(j), CHUNK)], sw[p]).wait()
            pltpu.make_async_copy(
                hdb[p], ohd_hbm.at[pl.ds(off(j), CHUNK)], sw[p]).wait()

        load_idx(0, 0)
        start_g(0)
        load_idx(1, 1)
        start_g(1)

        def start_addm(p):
            pltpu.async_copy(hs_hbm.at[si[p]], hdb[p], sg[p], add=True)

        def drain_addm(p):
            pltpu.make_async_copy(hs_hbm.at[si[p]], hdb[p], sg[p]).wait()

        def step(i, carry):
            j0 = 2 * i
            drain_g(0)
            start_addm(0)
            drain_g(1)
            start_addm(1)
            drain_addm(0)
            start_w(j0, 0)
            drain_addm(1)
            start_w(j0 + 1, 1)
            load_idx(j0 + 2, 0)
            drain_w(j0, 0)
            start_g(0)
            load_idx(j0 + 3, 1)
            drain_w(j0 + 1, 1)
            start_g(1)
            return carry

        lax.fori_loop(0, npairs - 1, step, 0)
        jl = 2 * (npairs - 1)
        drain_g(0)
        start_addm(0)
        drain_g(1)
        start_addm(1)
        drain_addm(0)
        start_w(jl, 0)
        drain_addm(1)
        start_w(jl + 1, 1)
        drain_w(jl, 0)
        drain_w(jl + 1, 1)

    return k(hs, hd, src, dst)


def _sc_scatter(wa2, dst, zeros):
    e = dst.shape[0]
    n = zeros.shape[0]
    nchunks = e // CHUNK
    rpt = n // NS
    mesh = plsc.VectorSubcoreMesh(core_axis_name="c", subcore_axis_name="s")

    @functools.partial(
        pl.kernel,
        out_type=jax.ShapeDtypeStruct((NC, n, HCOLS), F32),
        mesh=mesh,
        scratch_types=[
            pltpu.VMEM((CHUNK,), jnp.int32),
            pltpu.VMEM((CHUNK,), jnp.int32),
            pltpu.VMEM((CHUNK, HCOLS), F32),
            pltpu.VMEM((CHUNK, HCOLS), F32),
            pltpu.SemaphoreType.DMA,
            pltpu.SemaphoreType.DMA,
            pltpu.VMEM((rpt, HCOLS), F32),
            pltpu.VMEM_SHARED((n, HCOLS), F32),
        ],
        compiler_params=pltpu.CompilerParams(use_tc_tiling_on_sc=False),
    )
    def k(wa_hbm, dst_hbm, z_hbm, out_hbm,
          di0, di1, rows0, rows1, sr0, sr1, bounce, acc):
        c = lax.axis_index("c")
        s = lax.axis_index("s")
        r0 = s * rpt
        di = (di0, di1)
        rows = (rows0, rows1)
        sr = (sr0, sr1)

        pltpu.sync_copy(z_hbm.at[pl.ds(r0, rpt)], bounce)
        pltpu.sync_copy(bounce, acc.at[pl.ds(r0, rpt)])
        plsc.subcore_barrier()

        def off(j):
            return (s + j * NS) * CHUNK

        def start_r(j, p):
            pltpu.async_copy(dst_hbm.at[pl.ds(off(j), CHUNK)], di[p], sr[p])
            pltpu.async_copy(wa_hbm.at[c, pl.ds(off(j), CHUNK)], rows[p],
                             sr[p])

        def drain_r(j, p):
            pltpu.make_async_copy(
                dst_hbm.at[pl.ds(off(j), CHUNK)], di[p], sr[p]).wait()
            pltpu.make_async_copy(
                wa_hbm.at[c, pl.ds(off(j), CHUNK)], rows[p], sr[p]).wait()

        start_r(0, 0)
        start_r(1, 1)

        def step(i, carry):
            j0 = 2 * i
            drain_r(j0, 0)
            pltpu.sync_copy(rows[0], acc.at[di[0]], add=True)
            start_r(j0 + 2, 0)
            drain_r(j0 + 1, 1)
            pltpu.sync_copy(rows[1], acc.at[di[1]], add=True)
            start_r(j0 + 3, 1)
            return carry

        npairs = nchunks // NS // 2
        lax.fori_loop(0, npairs - 1, step, 0)
        jl = 2 * (npairs - 1)
        drain_r(jl, 0)
        pltpu.sync_copy(rows[0], acc.at[di[0]], add=True)
        drain_r(jl + 1, 1)
        pltpu.sync_copy(rows[1], acc.at[di[1]], add=True)
        plsc.subcore_barrier()

        pltpu.sync_copy(acc.at[pl.ds(r0, rpt)], bounce)
        pltpu.sync_copy(bounce, out_hbm.at[c, pl.ds(r0, rpt)])

    return k(wa2, dst, zeros)


def kernel(x_base, x_joint, x_foot, edge_index_bj, edge_index_jf,
           edge_index_fb, params):
    n = x_base.shape[0]
    e = edge_index_bj.shape[1]
    eb = e // 128

    npad = ((n + 8 * NS - 1) // (8 * NS)) * (8 * NS)
    if npad == n:
        npad += 8 * NS
    zeros_acc = jnp.zeros((npad, HCOLS), F32)

    ebp = ((eb + 2 * NS - 1) // (2 * NS)) * (2 * NS)
    ep = ebp * 128
    pad_dst = jnp.full((ep - e,), n, jnp.int32)

    hs1 = _linear_chain(x_base, params["enc_W_base"], params["enc_b_base"],
                        params["bj0_W_l"], params["bj0_b_l"])
    hd2 = _linear_chain(x_foot, params["enc_W_foot"], params["enc_b_foot"],
                        params["jf0_W_r"], params["jf0_b_r"])
    hj = _linear(x_joint, params["enc_W_joint"], params["enc_b_joint"], True)
    hd1, hs2 = _linear_dual(hj, params["bj0_W_r"], params["bj0_b_r"],
                            params["jf0_W_l"], params["jf0_b_l"])

    def agg2(hs, hd, edges, pre):
        src = edges[0]
        dst = edges[1]
        hs_g, m_g = _sc_gather2(hs, hd, src, dst)
        hs_g3 = hs_g.reshape(eb, 128, 128)
        m_g3 = m_g.reshape(eb, 128, 128)
        e2, bmax = _edge_scores(m_g3, params[pre + "_att"])
        wa = _edge_weights(e2, bmax, hs_g3, ebp)
        dst_p = jnp.concatenate([dst, pad_dst])
        return _sc_scatter(wa.reshape(2, ep, HCOLS), dst_p, zeros_acc)

    p1 = agg2(hs1, hd1, edge_index_bj, "bj0")
    hs3 = _finalize_linear(p1, params["bj0_bias"], params["jf1_W_l"],
                           params["jf1_b_l"], n)
    p2 = agg2(hs2, hd2, edge_index_jf, "jf0")
    hd3 = _finalize_linear(p2, params["jf0_bias"], params["jf1_W_r"],
                           params["jf1_b_r"], n)
    p3 = agg2(hs3, hd3, edge_index_jf, "jf1")

    out_dim = params["dec_W"].shape[1]
    dec_w = jnp.pad(params["dec_W"], ((0, 0), (0, 128 - out_dim)))
    dec_b = jnp.pad(params["dec_b"], (0, 128 - out_dim))
    out = _finalize_linear(p3, params["jf1_bias"], dec_w, dec_b, n)
    return out[:, :out_dim]


# speedup vs baseline: 1.1619x; 1.1619x over previous
import functools

import jax
import jax.numpy as jnp
from jax import lax
from jax.experimental import pallas as pl
from jax.experimental.pallas import tpu as pltpu
from jax.experimental.pallas import tpu_sc as plsc

F32 = jnp.float32

NC = 2
NS = 16
NW = NC * NS

CHUNK = 128
HCOLS = 80


def _linear(x, w, b, relu):
    n, d = x.shape
    dout = w.shape[1]
    blk = 2000

    def body(x_ref, w_ref, b_ref, o_ref):
        y = jnp.dot(x_ref[...], w_ref[...], preferred_element_type=F32)
        y = y + b_ref[...]
        if relu:
            y = jnp.maximum(y, 0.0)
        o_ref[...] = y

    return pl.pallas_call(
        body,
        grid=(n // blk,),
        in_specs=[
            pl.BlockSpec((blk, d), lambda i: (i, 0)),
            pl.BlockSpec((d, dout), lambda i: (0, 0)),
            pl.BlockSpec((1, dout), lambda i: (0, 0)),
        ],
        out_specs=pl.BlockSpec((blk, dout), lambda i: (i, 0)),
        out_shape=jax.ShapeDtypeStruct((n, dout), F32),
    )(x, w, b.reshape(1, dout))


def _linear_chain(x, w1, b1, w2, b2):
    n, d = x.shape
    dout = w2.shape[1]
    blk = 2000

    def body(x_ref, w1_ref, b1_ref, w2_ref, b2_ref, o_ref):
        y = jnp.dot(x_ref[...], w1_ref[...], preferred_element_type=F32)
        y = jnp.maximum(y + b1_ref[...], 0.0)
        o_ref[...] = jnp.dot(y, w2_ref[...],
                             preferred_element_type=F32) + b2_ref[...]

    h = w1.shape[1]
    return pl.pallas_call(
        body,
        grid=(n // blk,),
        in_specs=[
            pl.BlockSpec((blk, d), lambda i: (i, 0)),
            pl.BlockSpec((d, h), lambda i: (0, 0)),
            pl.BlockSpec((1, h), lambda i: (0, 0)),
            pl.BlockSpec((h, dout), lambda i: (0, 0)),
            pl.BlockSpec((1, dout), lambda i: (0, 0)),
        ],
        out_specs=pl.BlockSpec((blk, dout), lambda i: (i, 0)),
        out_shape=jax.ShapeDtypeStruct((n, dout), F32),
    )(x, w1, b1.reshape(1, h), w2, b2.reshape(1, dout))


def _linear_dual(x, wa, ba, wb, bb):
    n, d = x.shape
    dout = wa.shape[1]
    blk = 2000

    def body(x_ref, wa_ref, ba_ref, wb_ref, bb_ref, oa_ref, ob_ref):
        xv = x_ref[...]
        oa_ref[...] = jnp.dot(xv, wa_ref[...],
                              preferred_element_type=F32) + ba_ref[...]
        ob_ref[...] = jnp.dot(xv, wb_ref[...],
                              preferred_element_type=F32) + bb_ref[...]

    return pl.pallas_call(
        body,
        grid=(n // blk,),
        in_specs=[
            pl.BlockSpec((blk, d), lambda i: (i, 0)),
            pl.BlockSpec((d, dout), lambda i: (0, 0)),
            pl.BlockSpec((1, dout), lambda i: (0, 0)),
            pl.BlockSpec((d, dout), lambda i: (0, 0)),
            pl.BlockSpec((1, dout), lambda i: (0, 0)),
        ],
        out_specs=[
            pl.BlockSpec((blk, dout), lambda i: (i, 0)),
            pl.BlockSpec((blk, dout), lambda i: (i, 0)),
        ],
        out_shape=[
            jax.ShapeDtypeStruct((n, dout), F32),
            jax.ShapeDtypeStruct((n, dout), F32),
        ],
    )(x, wa, ba.reshape(1, dout), wb, bb.reshape(1, dout))


def _edge_scores(hs_g3, hd_g3, att):
    eb = hs_g3.shape[0]
    rb = 25
    g = eb // rb

    def body(hs_ref, hd_ref, att_ref, e_ref, bm_ref):
        m = hs_ref[...] + hd_ref[...]
        m = jnp.where(m > 0, m, 0.2 * m)
        a = att_ref[0, :]
        e = lax.dot_general(m, a, (((2,), (0,)), ((), ())),
                            preferred_element_type=F32)
        e_ref[...] = e[None]
        bm_ref[...] = jnp.full((1, 1, 128), jnp.max(e), F32)

    return pl.pallas_call(
        body,
        grid=(g,),
        in_specs=[
            pl.BlockSpec((rb, 128, 128), lambda i: (i, 0, 0)),
            pl.BlockSpec((rb, 128, 128), lambda i: (i, 0, 0)),
            pl.BlockSpec((1, 128), lambda i: (0, 0)),
        ],
        out_specs=[
            pl.BlockSpec((1, rb, 128), lambda i: (i, 0, 0)),
            pl.BlockSpec((1, 1, 128), lambda i: (i, 0, 0)),
        ],
        out_shape=[
            jax.ShapeDtypeStruct((g, rb, 128), F32),
            jax.ShapeDtypeStruct((g, 1, 128), F32),
        ],
    )(hs_g3, hd_g3, att.reshape(1, 128))


def _edge_weights(e2, bmax, hs_g3, ebp):
    eb = hs_g3.shape[0]
    rb = 25
    g = eb // rb

    def body(e_ref, bm_ref, hs_ref, wa_ref):
        mglob = jnp.max(bm_ref[...])
        ex = jnp.exp(e_ref[0] - mglob)
        hs = hs_ref[...]
        exn = ex[:, :, None]
        w = hs * exn
        z16 = jnp.zeros((rb, 128, 16), F32)
        i16 = lax.broadcasted_iota(jnp.int32, (rb, 128, 16), 2)
        x16 = jnp.where(i16 == 0, exn, 0.0)
        wa_ref[0] = jnp.concatenate([w[:, :, :64], z16], axis=2)
        wa_ref[1] = jnp.concatenate([w[:, :, 64:], x16], axis=2)

    return pl.pallas_call(
        body,
        grid=(g,),
        in_specs=[
            pl.BlockSpec((1, rb, 128), lambda i: (i, 0, 0)),
            pl.BlockSpec((g, 1, 128), lambda i: (0, 0, 0)),
            pl.BlockSpec((rb, 128, 128), lambda i: (i, 0, 0)),
        ],
        out_specs=pl.BlockSpec((2, rb, 128, HCOLS), lambda i: (0, i, 0, 0)),
        out_shape=jax.ShapeDtypeStruct((2, ebp, 128, HCOLS), F32),
    )(e2, bmax, hs_g3)


def _finalize_linear(p, bias, w, b, n):
    blk = 2000
    dout = w.shape[1]

    def body(p_ref, bias_ref, w_ref, b_ref, o_ref):
        lo = p_ref[0]
        hi = p_ref[1]
        u = jnp.concatenate([lo[:, :64], hi[:, :64]], axis=1)
        den = hi[:, 64:65]
        y = u / (den + 1e-16) + bias_ref[...]
        x = jnp.maximum(y, 0.0)
        o_ref[...] = jnp.dot(x, w_ref[...],
                             preferred_element_type=F32) + b_ref[...]

    return pl.pallas_call(
        body,
        grid=(n // blk,),
        in_specs=[
            pl.BlockSpec((2, blk, HCOLS), lambda i: (0, i, 0)),
            pl.BlockSpec((1, 128), lambda i: (0, 0)),
            pl.BlockSpec((128, dout), lambda i: (0, 0)),
            pl.BlockSpec((1, dout), lambda i: (0, 0)),
        ],
        out_specs=pl.BlockSpec((blk, dout), lambda i: (i, 0)),
        out_shape=jax.ShapeDtypeStruct((n, dout), F32),
    )(p, bias.reshape(1, 128), w, b.reshape(1, dout))


def _sc_gather2(hs, hd, src, dst):
    n, h = hs.shape
    e = src.shape[0]
    dt = hs.dtype
    nchunks = e // CHUNK
    nju = (nchunks + NW - 1) // NW
    nju = nju + (nju & 1)
    npairs = nju // 2
    mesh = plsc.VectorSubcoreMesh(core_axis_name="c", subcore_axis_name="s")

    @functools.partial(
        pl.kernel,
        out_type=(jax.ShapeDtypeStruct((e, h), dt),
                  jax.ShapeDtypeStruct((e, h), dt)),
        mesh=mesh,
        scratch_types=[
            pltpu.VMEM((CHUNK,), jnp.int32),
            pltpu.VMEM((CHUNK,), jnp.int32),
            pltpu.VMEM((CHUNK,), jnp.int32),
            pltpu.VMEM((CHUNK,), jnp.int32),
            pltpu.VMEM((CHUNK, h), dt),
            pltpu.VMEM((CHUNK, h), dt),
            pltpu.VMEM((CHUNK, h), dt),
            pltpu.VMEM((CHUNK, h), dt),
            pltpu.SemaphoreType.DMA,
            pltpu.SemaphoreType.DMA,
            pltpu.SemaphoreType.DMA,
            pltpu.SemaphoreType.DMA,
        ],
        compiler_params=pltpu.CompilerParams(use_tc_tiling_on_sc=False),
    )
    def k(hs_hbm, hd_hbm, src_hbm, dst_hbm, ohs_hbm, ohd_hbm,
          si0, si1, di0, di1, hsb0, hsb1, hdb0, hdb1, sg0, sg1, sw0, sw1):
        wid = lax.axis_index("s") * NC + lax.axis_index("c")
        si = (si0, si1)
        di = (di0, di1)
        hsb = (hsb0, hsb1)
        hdb = (hdb0, hdb1)
        sg = (sg0, sg1)
        sw = (sw0, sw1)
        nj = (nchunks - wid + NW - 1) // NW

        def off(j):
            jc = jnp.minimum(j, nj - 1)
            return (wid + jc * NW) * CHUNK

        def load_idx(j, p):
            pltpu.sync_copy(src_hbm.at[pl.ds(off(j), CHUNK)], si[p])
            pltpu.sync_copy(dst_hbm.at[pl.ds(off(j), CHUNK)], di[p])

        def start_g(p):
            pltpu.async_copy(hs_hbm.at[si[p]], hsb[p], sg[p])
            pltpu.async_copy(hd_hbm.at[di[p]], hdb[p], sg[p])

        def drain_g(p):
            pltpu.make_async_copy(hs_hbm.at[si[p]], hsb[p], sg[p]).wait()
            pltpu.make_async_copy(hd_hbm.at[di[p]], hdb[p], sg[p]).wait()

        def start_w(j, p):
            pltpu.async_copy(hsb[p], ohs_hbm.at[pl.ds(off(j), CHUNK)], sw[p])
            pltpu.async_copy(hdb[p], ohd_hbm.at[pl.ds(off(j), CHUNK)], sw[p])

        def drain_w(j, p):
            pltpu.make_async_copy(
                hsb[p], ohs_hbm.at[pl.ds(off(j), CHUNK)], sw[p]).wait()
            pltpu.make_async_copy(
                hdb[p], ohd_hbm.at[pl.ds(off(j), CHUNK)], sw[p]).wait()

        load_idx(0, 0)
        start_g(0)
        load_idx(1, 1)
        start_g(1)

        def step(i, carry):
            j0 = 2 * i
            drain_g(0)
            start_w(j0, 0)
            drain_g(1)
            start_w(j0 + 1, 1)
            load_idx(j0 + 2, 0)
            drain_w(j0, 0)
            start_g(0)
            load_idx(j0 + 3, 1)
            drain_w(j0 + 1, 1)
            start_g(1)
            return carry

        lax.fori_loop(0, npairs - 1, step, 0)
        jl = 2 * (npairs - 1)
        drain_g(0)
        start_w(jl, 0)
        drain_g(1)
        start_w(jl + 1, 1)
        drain_w(jl, 0)
        drain_w(jl + 1, 1)

    return k(hs, hd, src, dst)


def _sc_scatter(wa2, dst, zeros):
    e = dst.shape[0]
    n = zeros.shape[0]
    nchunks = e // CHUNK
    rpt = n // NS
    mesh = plsc.VectorSubcoreMesh(core_axis_name="c", subcore_axis_name="s")

    @functools.partial(
        pl.kernel,
        out_type=jax.ShapeDtypeStruct((NC, n, HCOLS), F32),
        mesh=mesh,
        scratch_types=[
            pltpu.VMEM((CHUNK,), jnp.int32),
            pltpu.VMEM((CHUNK,), jnp.int32),
            pltpu.VMEM((CHUNK, HCOLS), F32),
            pltpu.VMEM((CHUNK, HCOLS), F32),
            pltpu.SemaphoreType.DMA,
            pltpu.SemaphoreType.DMA,
            pltpu.VMEM((rpt, HCOLS), F32),
            pltpu.VMEM_SHARED((n, HCOLS), F32),
        ],
        compiler_params=pltpu.CompilerParams(use_tc_tiling_on_sc=False),
    )
    def k(wa_hbm, dst_hbm, z_hbm, out_hbm,
          di0, di1, rows0, rows1, sr0, sr1, bounce, acc):
        c = lax.axis_index("c")
        s = lax.axis_index("s")
        r0 = s * rpt
        di = (di0, di1)
        rows = (rows0, rows1)
        sr = (sr0, sr1)

        pltpu.sync_copy(z_hbm.at[pl.ds(r0, rpt)], bounce)
        pltpu.sync_copy(bounce, acc.at[pl.ds(r0, rpt)])
        plsc.subcore_barrier()

        def off(j):
            return (s + j * NS) * CHUNK

        def start_r(j, p):
            pltpu.async_copy(dst_hbm.at[pl.ds(off(j), CHUNK)], di[p], sr[p])
            pltpu.async_copy(wa_hbm.at[c, pl.ds(off(j), CHUNK)], rows[p],
                             sr[p])

        def drain_r(j, p):
            pltpu.make_async_copy(
                dst_hbm.at[pl.ds(off(j), CHUNK)], di[p], sr[p]).wait()
            pltpu.make_async_copy(
                wa_hbm.at[c, pl.ds(off(j), CHUNK)], rows[p], sr[p]).wait()

        start_r(0, 0)
        start_r(1, 1)

        def step(i, carry):
            j0 = 2 * i
            drain_r(j0, 0)
            pltpu.sync_copy(rows[0], acc.at[di[0]], add=True)
            start_r(j0 + 2, 0)
            drain_r(j0 + 1, 1)
            pltpu.sync_copy(rows[1], acc.at[di[1]], add=True)
            start_r(j0 + 3, 1)
            return carry

        npairs = nchunks // NS // 2
        lax.fori_loop(0, npairs - 1, step, 0)
        jl = 2 * (npairs - 1)
        drain_r(jl, 0)
        pltpu.sync_copy(rows[0], acc.at[di[0]], add=True)
        drain_r(jl + 1, 1)
        pltpu.sync_copy(rows[1], acc.at[di[1]], add=True)
        plsc.subcore_barrier()

        pltpu.sync_copy(acc.at[pl.ds(r0, rpt)], bounce)
        pltpu.sync_copy(bounce, out_hbm.at[c, pl.ds(r0, rpt)])

    return k(wa2, dst, zeros)


def kernel(x_base, x_joint, x_foot, edge_index_bj, edge_index_jf,
           edge_index_fb, params):
    n = x_base.shape[0]
    e = edge_index_bj.shape[1]
    eb = e // 128

    npad = ((n + 8 * NS - 1) // (8 * NS)) * (8 * NS)
    if npad == n:
        npad += 8 * NS
    zeros_acc = jnp.zeros((npad, HCOLS), F32)

    ebp = ((eb + 2 * NS - 1) // (2 * NS)) * (2 * NS)
    ep = ebp * 128
    pad_dst = jnp.full((ep - e,), n, jnp.int32)

    hs1 = _linear_chain(x_base, params["enc_W_base"], params["enc_b_base"],
                        params["bj0_W_l"], params["bj0_b_l"])
    hd2 = _linear_chain(x_foot, params["enc_W_foot"], params["enc_b_foot"],
                        params["jf0_W_r"], params["jf0_b_r"])
    hj = _linear(x_joint, params["enc_W_joint"], params["enc_b_joint"], True)
    hd1, hs2 = _linear_dual(hj, params["bj0_W_r"], params["bj0_b_r"],
                            params["jf0_W_l"], params["jf0_b_l"])

    def agg2(hs, hd, edges, pre):
        src = edges[0]
        dst = edges[1]
        hs_g, hd_g = _sc_gather2(hs, hd, src, dst)
        hs_g3 = hs_g.reshape(eb, 128, 128)
        hd_g3 = hd_g.reshape(eb, 128, 128)
        e2, bmax = _edge_scores(hs_g3, hd_g3, params[pre + "_att"])
        wa = _edge_weights(e2, bmax, hs_g3, ebp)
        dst_p = jnp.concatenate([dst, pad_dst])
        return _sc_scatter(wa.reshape(2, ep, HCOLS), dst_p, zeros_acc)

    p1 = agg2(hs1, hd1, edge_index_bj, "bj0")
    hs3 = _finalize_linear(p1, params["bj0_bias"], params["jf1_W_l"],
                           params["jf1_b_l"], n)
    p2 = agg2(hs2, hd2, edge_index_jf, "jf0")
    hd3 = _finalize_linear(p2, params["jf0_bias"], params["jf1_W_r"],
                           params["jf1_b_r"], n)
    p3 = agg2(hs3, hd3, edge_index_jf, "jf1")

    out_dim = params["dec_W"].shape[1]
    dec_w = jnp.pad(params["dec_W"], ((0, 0), (0, 128 - out_dim)))
    dec_b = jnp.pad(params["dec_b"], (0, 128 - out_dim))
    out = _finalize_linear(p3, params["jf1_bias"], dec_w, dec_b, n)
    return out[:, :out_dim]


# speedup vs baseline: 1.2161x; 1.0467x over previous
import functools

import jax
import jax.numpy as jnp
from jax import lax
from jax.experimental import pallas as pl
from jax.experimental.pallas import tpu as pltpu
from jax.experimental.pallas import tpu_sc as plsc

F32 = jnp.float32

NC = 2
NS = 16
NW = NC * NS

CHUNK = 128
HCOLS = 80


def _linear(x, w, b, relu):
    n, d = x.shape
    dout = w.shape[1]
    blk = 5000

    def body(x_ref, w_ref, b_ref, o_ref):
        y = jnp.dot(x_ref[...], w_ref[...], preferred_element_type=F32)
        y = y + b_ref[...]
        if relu:
            y = jnp.maximum(y, 0.0)
        o_ref[...] = y

    return pl.pallas_call(
        body,
        grid=(n // blk,),
        in_specs=[
            pl.BlockSpec((blk, d), lambda i: (i, 0)),
            pl.BlockSpec((d, dout), lambda i: (0, 0)),
            pl.BlockSpec((1, dout), lambda i: (0, 0)),
        ],
        out_specs=pl.BlockSpec((blk, dout), lambda i: (i, 0)),
        out_shape=jax.ShapeDtypeStruct((n, dout), F32),
    )(x, w, b.reshape(1, dout))


def _linear_chain(x, w1, b1, w2, b2):
    n, d = x.shape
    dout = w2.shape[1]
    blk = 5000

    def body(x_ref, w1_ref, b1_ref, w2_ref, b2_ref, o_ref):
        y = jnp.dot(x_ref[...], w1_ref[...], preferred_element_type=F32)
        y = jnp.maximum(y + b1_ref[...], 0.0)
        o_ref[...] = jnp.dot(y, w2_ref[...],
                             preferred_element_type=F32) + b2_ref[...]

    h = w1.shape[1]
    return pl.pallas_call(
        body,
        grid=(n // blk,),
        in_specs=[
            pl.BlockSpec((blk, d), lambda i: (i, 0)),
            pl.BlockSpec((d, h), lambda i: (0, 0)),
            pl.BlockSpec((1, h), lambda i: (0, 0)),
            pl.BlockSpec((h, dout), lambda i: (0, 0)),
            pl.BlockSpec((1, dout), lambda i: (0, 0)),
        ],
        out_specs=pl.BlockSpec((blk, dout), lambda i: (i, 0)),
        out_shape=jax.ShapeDtypeStruct((n, dout), F32),
    )(x, w1, b1.reshape(1, h), w2, b2.reshape(1, dout))


def _linear_dual(x, wa, ba, wb, bb):
    n, d = x.shape
    dout = wa.shape[1]
    blk = 5000

    def body(x_ref, wa_ref, ba_ref, wb_ref, bb_ref, oa_ref, ob_ref):
        xv = x_ref[...]
        oa_ref[...] = jnp.dot(xv, wa_ref[...],
                              preferred_element_type=F32) + ba_ref[...]
        ob_ref[...] = jnp.dot(xv, wb_ref[...],
                              preferred_element_type=F32) + bb_ref[...]

    return pl.pallas_call(
        body,
        grid=(n // blk,),
        in_specs=[
            pl.BlockSpec((blk, d), lambda i: (i, 0)),
            pl.BlockSpec((d, dout), lambda i: (0, 0)),
            pl.BlockSpec((1, dout), lambda i: (0, 0)),
            pl.BlockSpec((d, dout), lambda i: (0, 0)),
            pl.BlockSpec((1, dout), lambda i: (0, 0)),
        ],
        out_specs=[
            pl.BlockSpec((blk, dout), lambda i: (i, 0)),
            pl.BlockSpec((blk, dout), lambda i: (i, 0)),
        ],
        out_shape=[
            jax.ShapeDtypeStruct((n, dout), F32),
            jax.ShapeDtypeStruct((n, dout), F32),
        ],
    )(x, wa, ba.reshape(1, dout), wb, bb.reshape(1, dout))


def _edge_scores(hs_g3, hd_g3, att):
    eb = hs_g3.shape[0]
    rb = 50
    g = eb // rb

    def body(hs_ref, hd_ref, att_ref, e_ref, bm_ref):
        m = hs_ref[...] + hd_ref[...]
        m = jnp.where(m > 0, m, 0.2 * m)
        a = att_ref[0, :]
        e = lax.dot_general(m, a, (((2,), (0,)), ((), ())),
                            preferred_element_type=F32)
        e_ref[...] = e[None]
        bm_ref[...] = jnp.full((1, 1, 128), jnp.max(e), F32)

    return pl.pallas_call(
        body,
        grid=(g,),
        in_specs=[
            pl.BlockSpec((rb, 128, 128), lambda i: (i, 0, 0)),
            pl.BlockSpec((rb, 128, 128), lambda i: (i, 0, 0)),
            pl.BlockSpec((1, 128), lambda i: (0, 0)),
        ],
        out_specs=[
            pl.BlockSpec((1, rb, 128), lambda i: (i, 0, 0)),
            pl.BlockSpec((1, 1, 128), lambda i: (i, 0, 0)),
        ],
        out_shape=[
            jax.ShapeDtypeStruct((g, rb, 128), F32),
            jax.ShapeDtypeStruct((g, 1, 128), F32),
        ],
    )(hs_g3, hd_g3, att.reshape(1, 128))


def _edge_weights(e2, bmax, hs_g3, ebp):
    eb = hs_g3.shape[0]
    rb = 50
    g = eb // rb

    def body(e_ref, bm_ref, hs_ref, wa_ref):
        mglob = jnp.max(bm_ref[...])
        ex = jnp.exp(e_ref[0] - mglob)
        hs = hs_ref[...]
        exn = ex[:, :, None]
        w = hs * exn
        z16 = jnp.zeros((rb, 128, 16), F32)
        i16 = lax.broadcasted_iota(jnp.int32, (rb, 128, 16), 2)
        x16 = jnp.where(i16 == 0, exn, 0.0)
        wa_ref[0] = jnp.concatenate([w[:, :, :64], z16], axis=2)
        wa_ref[1] = jnp.concatenate([w[:, :, 64:], x16], axis=2)

    return pl.pallas_call(
        body,
        grid=(g,),
        in_specs=[
            pl.BlockSpec((1, rb, 128), lambda i: (i, 0, 0)),
            pl.BlockSpec((g, 1, 128), lambda i: (0, 0, 0)),
            pl.BlockSpec((rb, 128, 128), lambda i: (i, 0, 0)),
        ],
        out_specs=pl.BlockSpec((2, rb, 128, HCOLS), lambda i: (0, i, 0, 0)),
        out_shape=jax.ShapeDtypeStruct((2, ebp, 128, HCOLS), F32),
    )(e2, bmax, hs_g3)


def _finalize_linear(p, bias, w, b, n):
    blk = 5000
    dout = w.shape[1]

    def body(p_ref, bias_ref, w_ref, b_ref, o_ref):
        lo = p_ref[0]
        hi = p_ref[1]
        u = jnp.concatenate([lo[:, :64], hi[:, :64]], axis=1)
        den = hi[:, 64:65]
        y = u / (den + 1e-16) + bias_ref[...]
        x = jnp.maximum(y, 0.0)
        o_ref[...] = jnp.dot(x, w_ref[...],
                             preferred_element_type=F32) + b_ref[...]

    return pl.pallas_call(
        body,
        grid=(n // blk,),
        in_specs=[
            pl.BlockSpec((2, blk, HCOLS), lambda i: (0, i, 0)),
            pl.BlockSpec((1, 128), lambda i: (0, 0)),
            pl.BlockSpec((128, dout), lambda i: (0, 0)),
            pl.BlockSpec((1, dout), lambda i: (0, 0)),
        ],
        out_specs=pl.BlockSpec((blk, dout), lambda i: (i, 0)),
        out_shape=jax.ShapeDtypeStruct((n, dout), F32),
    )(p, bias.reshape(1, 128), w, b.reshape(1, dout))


def _sc_gather2(hs, hd, src, dst):
    n, h = hs.shape
    e = src.shape[0]
    dt = hs.dtype
    nchunks = e // CHUNK
    nju = (nchunks + NW - 1) // NW
    nju = nju + (nju & 1)
    npairs = nju // 2
    mesh = plsc.VectorSubcoreMesh(core_axis_name="c", subcore_axis_name="s")

    @functools.partial(
        pl.kernel,
        out_type=(jax.ShapeDtypeStruct((e, h), dt),
                  jax.ShapeDtypeStruct((e, h), dt)),
        mesh=mesh,
        scratch_types=[
            pltpu.VMEM((CHUNK,), jnp.int32),
            pltpu.VMEM((CHUNK,), jnp.int32),
            pltpu.VMEM((CHUNK,), jnp.int32),
            pltpu.VMEM((CHUNK,), jnp.int32),
            pltpu.VMEM((CHUNK, h), dt),
            pltpu.VMEM((CHUNK, h), dt),
            pltpu.VMEM((CHUNK, h), dt),
            pltpu.VMEM((CHUNK, h), dt),
            pltpu.SemaphoreType.DMA,
            pltpu.SemaphoreType.DMA,
            pltpu.SemaphoreType.DMA,
            pltpu.SemaphoreType.DMA,
        ],
        compiler_params=pltpu.CompilerParams(use_tc_tiling_on_sc=False),
    )
    def k(hs_hbm, hd_hbm, src_hbm, dst_hbm, ohs_hbm, ohd_hbm,
          si0, si1, di0, di1, hsb0, hsb1, hdb0, hdb1, sg0, sg1, sw0, sw1):
        wid = lax.axis_index("s") * NC + lax.axis_index("c")
        si = (si0, si1)
        di = (di0, di1)
        hsb = (hsb0, hsb1)
        hdb = (hdb0, hdb1)
        sg = (sg0, sg1)
        sw = (sw0, sw1)
        nj = (nchunks - wid + NW - 1) // NW

        def off(j):
            jc = jnp.minimum(j, nj - 1)
            return (wid + jc * NW) * CHUNK

        def load_idx(j, p):
            pltpu.sync_copy(src_hbm.at[pl.ds(off(j), CHUNK)], si[p])
            pltpu.sync_copy(dst_hbm.at[pl.ds(off(j), CHUNK)], di[p])

        def start_g(p):
            pltpu.async_copy(hs_hbm.at[si[p]], hsb[p], sg[p])
            pltpu.async_copy(hd_hbm.at[di[p]], hdb[p], sg[p])

        def drain_g(p):
            pltpu.make_async_copy(hs_hbm.at[si[p]], hsb[p], sg[p]).wait()
            pltpu.make_async_copy(hd_hbm.at[di[p]], hdb[p], sg[p]).wait()

        def start_w(j, p):
            pltpu.async_copy(hsb[p], ohs_hbm.at[pl.ds(off(j), CHUNK)], sw[p])
            pltpu.async_copy(hdb[p], ohd_hbm.at[pl.ds(off(j), CHUNK)], sw[p])

        def drain_w(j, p):
            pltpu.make_async_copy(
                hsb[p], ohs_hbm.at[pl.ds(off(j), CHUNK)], sw[p]).wait()
            pltpu.make_async_copy(
                hdb[p], ohd_hbm.at[pl.ds(off(j), CHUNK)], sw[p]).wait()

        load_idx(0, 0)
        start_g(0)
        load_idx(1, 1)
        start_g(1)

        def step(i, carry):
            j0 = 2 * i
            drain_g(0)
            start_w(j0, 0)
            drain_g(1)
            start_w(j0 + 1, 1)
            load_idx(j0 + 2, 0)
            drain_w(j0, 0)
            start_g(0)
            load_idx(j0 + 3, 1)
            drain_w(j0 + 1, 1)
            start_g(1)
            return carry

        lax.fori_loop(0, npairs - 1, step, 0)
        jl = 2 * (npairs - 1)
        drain_g(0)
        start_w(jl, 0)
        drain_g(1)
        start_w(jl + 1, 1)
        drain_w(jl, 0)
        drain_w(jl + 1, 1)

    return k(hs, hd, src, dst)


def _sc_scatter(wa2, dst, zeros):
    e = dst.shape[0]
    n = zeros.shape[0]
    nchunks = e // CHUNK
    rpt = n // NS
    mesh = plsc.VectorSubcoreMesh(core_axis_name="c", subcore_axis_name="s")

    @functools.partial(
        pl.kernel,
        out_type=jax.ShapeDtypeStruct((NC, n, HCOLS), F32),
        mesh=mesh,
        scratch_types=[
            pltpu.VMEM((CHUNK,), jnp.int32),
            pltpu.VMEM((CHUNK,), jnp.int32),
            pltpu.VMEM((CHUNK, HCOLS), F32),
            pltpu.VMEM((CHUNK, HCOLS), F32),
            pltpu.SemaphoreType.DMA,
            pltpu.SemaphoreType.DMA,
            pltpu.VMEM((rpt, HCOLS), F32),
            pltpu.VMEM_SHARED((n, HCOLS), F32),
        ],
        compiler_params=pltpu.CompilerParams(use_tc_tiling_on_sc=False),
    )
    def k(wa_hbm, dst_hbm, z_hbm, out_hbm,
          di0, di1, rows0, rows1, sr0, sr1, bounce, acc):
        c = lax.axis_index("c")
        s = lax.axis_index("s")
        r0 = s * rpt
        di = (di0, di1)
        rows = (rows0, rows1)
        sr = (sr0, sr1)

        pltpu.sync_copy(z_hbm.at[pl.ds(r0, rpt)], bounce)
        pltpu.sync_copy(bounce, acc.at[pl.ds(r0, rpt)])
        plsc.subcore_barrier()

        def off(j):
            return (s + j * NS) * CHUNK

        def start_r(j, p):
            pltpu.async_copy(dst_hbm.at[pl.ds(off(j), CHUNK)], di[p], sr[p])
            pltpu.async_copy(wa_hbm.at[c, pl.ds(off(j), CHUNK)], rows[p],
                             sr[p])

        def drain_r(j, p):
            pltpu.make_async_copy(
                dst_hbm.at[pl.ds(off(j), CHUNK)], di[p], sr[p]).wait()
            pltpu.make_async_copy(
                wa_hbm.at[c, pl.ds(off(j), CHUNK)], rows[p], sr[p]).wait()

        start_r(0, 0)
        start_r(1, 1)

        def step(i, carry):
            j0 = 2 * i
            drain_r(j0, 0)
            pltpu.sync_copy(rows[0], acc.at[di[0]], add=True)
            start_r(j0 + 2, 0)
            drain_r(j0 + 1, 1)
            pltpu.sync_copy(rows[1], acc.at[di[1]], add=True)
            start_r(j0 + 3, 1)
            return carry

        npairs = nchunks // NS // 2
        lax.fori_loop(0, npairs - 1, step, 0)
        jl = 2 * (npairs - 1)
        drain_r(jl, 0)
        pltpu.sync_copy(rows[0], acc.at[di[0]], add=True)
        drain_r(jl + 1, 1)
        pltpu.sync_copy(rows[1], acc.at[di[1]], add=True)
        plsc.subcore_barrier()

        pltpu.sync_copy(acc.at[pl.ds(r0, rpt)], bounce)
        pltpu.sync_copy(bounce, out_hbm.at[c, pl.ds(r0, rpt)])

    return k(wa2, dst, zeros)


def kernel(x_base, x_joint, x_foot, edge_index_bj, edge_index_jf,
           edge_index_fb, params):
    n = x_base.shape[0]
    e = edge_index_bj.shape[1]
    eb = e // 128

    npad = ((n + 8 * NS - 1) // (8 * NS)) * (8 * NS)
    if npad == n:
        npad += 8 * NS
    zeros_acc = jnp.zeros((npad, HCOLS), F32)

    ebp = ((eb + 2 * NS - 1) // (2 * NS)) * (2 * NS)
    ep = ebp * 128
    pad_dst = jnp.full((ep - e,), n, jnp.int32)

    hs1 = _linear_chain(x_base, params["enc_W_base"], params["enc_b_base"],
                        params["bj0_W_l"], params["bj0_b_l"])
    hd2 = _linear_chain(x_foot, params["enc_W_foot"], params["enc_b_foot"],
                        params["jf0_W_r"], params["jf0_b_r"])
    hj = _linear(x_joint, params["enc_W_joint"], params["enc_b_joint"], True)
    hd1, hs2 = _linear_dual(hj, params["bj0_W_r"], params["bj0_b_r"],
                            params["jf0_W_l"], params["jf0_b_l"])

    def agg2(hs, hd, edges, pre):
        src = edges[0]
        dst = edges[1]
        hs_g, hd_g = _sc_gather2(hs, hd, src, dst)
        hs_g3 = hs_g.reshape(eb, 128, 128)
        hd_g3 = hd_g.reshape(eb, 128, 128)
        e2, bmax = _edge_scores(hs_g3, hd_g3, params[pre + "_att"])
        wa = _edge_weights(e2, bmax, hs_g3, ebp)
        dst_p = jnp.concatenate([dst, pad_dst])
        return _sc_scatter(wa.reshape(2, ep, HCOLS), dst_p, zeros_acc)

    p1 = agg2(hs1, hd1, edge_index_bj, "bj0")
    hs3 = _finalize_linear(p1, params["bj0_bias"], params["jf1_W_l"],
                           params["jf1_b_l"], n)
    p2 = agg2(hs2, hd2, edge_index_jf, "jf0")
    hd3 = _finalize_linear(p2, params["jf0_bias"], params["jf1_W_r"],
                           params["jf1_b_r"], n)
    p3 = agg2(hs3, hd3, edge_index_jf, "jf1")

    out_dim = params["dec_W"].shape[1]
    dec_w = jnp.pad(params["dec_W"], ((0, 0), (0, 128 - out_dim)))
    dec_b = jnp.pad(params["dec_b"], (0, 128 - out_dim))
    out = _finalize_linear(p3, params["jf1_bias"], dec_w, dec_b, n)
    return out[:, :out_dim]
